# Initial kernel scaffold; baseline (speedup 1.0000x reference)
import jax
import jax.numpy as jnp
from jax.experimental import pallas as pl


def kernel(x, y, params):
    raise NotImplementedError("write your pallas kernel here")


# speedup vs baseline: 4.4893x; 4.4893x over previous
import functools

import jax
import jax.numpy as jnp
from jax.experimental import pallas as pl

K, EPS = 20, 1e-5
_NEG = -3.0e38


def _topk_body(inner_ref, xxr_ref, xxc_ref, idx_ref):
    b = pl.program_id(0)
    n = inner_ref.shape[-1]
    rb = inner_ref.shape[-2]
    nd = 2.0 * inner_ref[0] - xxr_ref[0] - xxc_ref[0]
    lanes = jax.lax.broadcasted_iota(jnp.int32, (rb, n), 1)
    lanes20 = jax.lax.broadcasted_iota(jnp.int32, (rb, K), 1)
    idxmat = jnp.zeros((rb, K), jnp.int32)
    for t in range(K):
        m = jnp.max(nd, axis=1, keepdims=True)
        ii = jnp.min(jnp.where(nd == m, lanes, n), axis=1, keepdims=True)
        idxmat = jnp.where(lanes20 == t, ii, idxmat)
        nd = jnp.where(lanes == ii, _NEG, nd)
    idx_ref[0] = idxmat + b * n


def _topk20(inner, xx):
    B, N, _ = inner.shape
    RB = 256
    xxr = xx.reshape(B, N, 1)
    xxc = xx.reshape(B, 1, N)
    idx = pl.pallas_call(
        _topk_body,
        grid=(B, N // RB),
        in_specs=[pl.BlockSpec((1, RB, N), lambda b, i: (b, i, 0)),
                  pl.BlockSpec((1, RB, 1), lambda b, i: (b, i, 0)),
                  pl.BlockSpec((1, 1, N), lambda b, i: (b, 0, 0))],
        out_specs=pl.BlockSpec((1, RB, K), lambda b, i: (b, i, 0)),
        out_shape=jax.ShapeDtypeStruct((B, N, K), jnp.int32),
    )(inner, xxr, xxc)
    return idx.reshape(B * N, K)


def _knn_flat(x):
    xx = jnp.sum(x * x, axis=1)
    inner = jnp.einsum('bdn,bdm->bnm', x, x)
    return _topk20(inner, xx)


def _conv_bn_lrelu(y, W, g, b):
    y = jnp.einsum('...c,oc->...o', y, W)
    axes = tuple(range(y.ndim - 1))
    mu = jnp.mean(y, axis=axes, keepdims=True)
    var = jnp.var(y, axis=axes, keepdims=True)
    y = (y - mu) / jnp.sqrt(var + EPS)
    y = y * g + b
    return jnp.where(y > 0, y, 0.2 * y)


def _graph_feature_flat(x, idxf):
    B, d, N = x.shape
    xt = jnp.transpose(x, (0, 2, 1)).reshape(B * N, d)
    nbrs = xt[idxf]
    center = xt[:, None, :]
    feat = jnp.concatenate([nbrs - center, jnp.broadcast_to(center, nbrs.shape)],
                           axis=-1)
    return feat.reshape(B, N, K, 2 * d)


def _edge_conv(x, W, g, b):
    idxf = _knn_flat(x)
    feat = _graph_feature_flat(x, idxf)
    y = _conv_bn_lrelu(feat, W, g, b)
    y = jnp.max(y, axis=2)
    return jnp.transpose(y, (0, 2, 1))


def _multi_edge_conv(x, layers):
    idxf = _knn_flat(x)
    feat = _graph_feature_flat(x, idxf)
    for (W, g, b) in layers:
        feat = _conv_bn_lrelu(feat, W, g, b)
    y = jnp.max(feat, axis=2)
    return jnp.transpose(y, (0, 2, 1))


def _encoder_fn(x, p):
    x1 = _multi_edge_conv(x, [(p['enc0_W0'], p['enc0_g0'], p['enc0_b0']),
                              (p['enc0_W1'], p['enc0_g1'], p['enc0_b1'])])
    x2 = _multi_edge_conv(x1, [(p['enc1_W0'], p['enc1_g0'], p['enc1_b0']),
                               (p['enc1_W1'], p['enc1_g1'], p['enc1_b1'])])
    x3 = _edge_conv(x2, p['enc2_W'], p['enc2_g'], p['enc2_b'])
    x4 = _edge_conv(x3, p['enc3_W'], p['enc3_g'], p['enc3_b'])
    return jnp.concatenate([x1, x2, x3, x4], axis=1)


def _tail_fn(x, p):
    y = jnp.einsum('bcn,oc->bon', x, p['tail_W'])
    mu = jnp.mean(y, axis=(0, 2), keepdims=True)
    var = jnp.var(y, axis=(0, 2), keepdims=True)
    y = (y - mu) / jnp.sqrt(var + EPS)
    y = y * p['tail_g'][None, :, None] + p['tail_b'][None, :, None]
    return jnp.where(y > 0, y, 0.2 * y)


def kernel(x, y, params):
    p = params
    x1 = _tail_fn(_encoder_fn(x, p), p)
    x2 = _tail_fn(_encoder_fn(y, p), p)
    z = jnp.concatenate([x1, x2], axis=1)
    z = _edge_conv(z, p['dec0_W'], p['dec0_g'], p['dec0_b'])
    z = _edge_conv(z, p['dec1_W'], p['dec1_g'], p['dec1_b'])
    return jnp.einsum('bcn,oc->bon', z, p['dec2_W']) + p['dec2_bias'][None, :, None]


# speedup vs baseline: 4.5645x; 1.0167x over previous
import functools

import jax
import jax.numpy as jnp
from jax.experimental import pallas as pl
from jax.experimental.pallas import tpu as pltpu
from jax.experimental.pallas import tpu_sc as plsc

K, EPS = 20, 1e-5
_NEG = -3.0e38


def _topk_body(inner_ref, xxr_ref, xxc_ref, idx_ref):
    b = pl.program_id(0)
    n = inner_ref.shape[-1]
    rb = inner_ref.shape[-2]
    nd = 2.0 * inner_ref[0] - xxr_ref[0] - xxc_ref[0]
    lanes = jax.lax.broadcasted_iota(jnp.int32, (rb, n), 1)
    lanes20 = jax.lax.broadcasted_iota(jnp.int32, (rb, K), 1)
    idxmat = jnp.zeros((rb, K), jnp.int32)
    for t in range(K):
        m = jnp.max(nd, axis=1, keepdims=True)
        ii = jnp.min(jnp.where(nd == m, lanes, n), axis=1, keepdims=True)
        idxmat = jnp.where(lanes20 == t, ii, idxmat)
        nd = jnp.where(lanes == ii, _NEG, nd)
    idx_ref[0] = idxmat + b * n


def _topk20(inner, xx):
    B, N, _ = inner.shape
    RB = 256
    xxr = xx.reshape(B, N, 1)
    xxc = xx.reshape(B, 1, N)
    idx = pl.pallas_call(
        _topk_body,
        grid=(B, N // RB),
        in_specs=[pl.BlockSpec((1, RB, N), lambda b, i: (b, i, 0)),
                  pl.BlockSpec((1, RB, 1), lambda b, i: (b, i, 0)),
                  pl.BlockSpec((1, 1, N), lambda b, i: (b, 0, 0))],
        out_specs=pl.BlockSpec((1, RB, K), lambda b, i: (b, i, 0)),
        out_shape=jax.ShapeDtypeStruct((B, N, K), jnp.int32),
    )(inner, xxr, xxc)
    return idx.reshape(B * N, K)


def _knn_flat(x):
    xx = jnp.sum(x * x, axis=1)
    inner = jnp.einsum('bdn,bdm->bnm', x, x)
    return _topk20(inner, xx)


@functools.lru_cache(maxsize=None)
def _sc_feat_fn(R, d, E):
    dp = ((d + 127) // 128) * 128
    info = plsc.get_sparse_core_info()
    NW = info.num_cores * info.num_subcores
    P_C = 4 if d <= 64 else 2
    CH_E = P_C * K
    NG = CH_E + 16
    EPW = E // NW
    n_chunks = EPW // CH_E
    assert EPW % CH_E == 0 and E % NW == 0 and CH_E % 8 == 0
    mesh = plsc.VectorSubcoreMesh(core_axis_name="c", subcore_axis_name="s")

    @functools.partial(
        pl.kernel, mesh=mesh,
        out_type=jax.ShapeDtypeStruct((E, 2 * d), jnp.float32),
        scratch_types=[
            pltpu.VMEM((NG,), jnp.int32),
            pltpu.VMEM((NG, dp), jnp.float32),
            pltpu.VMEM((CH_E, 2 * d), jnp.float32),
            pltpu.SemaphoreType.DMA,
        ],
    )
    def body(xt_hbm, idxaug_hbm, feat_hbm, idx_v, nbr_v, feat_v, sem):
        wid = jax.lax.axis_index("s") * info.num_cores + jax.lax.axis_index("c")

        def chunk(c, carry):
            base_e = wid * EPW + c * CH_E
            g = wid * n_chunks + c
            pltpu.sync_copy(idxaug_hbm.at[pl.ds(g * NG, NG)], idx_v)
            pltpu.async_copy(xt_hbm.at[idx_v], nbr_v, sem).wait()
            for p in range(P_C):
                for j in range(K):
                    r = p * K + j
                    for l in range(d // 16):
                        nv = nbr_v[r, pl.ds(l * 16, 16)]
                        cv = nbr_v[CH_E + p, pl.ds(l * 16, 16)]
                        feat_v[r, pl.ds(l * 16, 16)] = nv - cv
                        feat_v[r, pl.ds(d + l * 16, 16)] = cv
            pltpu.sync_copy(feat_v, feat_hbm.at[pl.ds(base_e, CH_E)])
            return carry

        jax.lax.fori_loop(0, n_chunks, chunk, 0)

    return body


@functools.lru_cache(maxsize=None)
def _sc_chunk_meta(R, d, E):
    info = plsc.get_sparse_core_info()
    NW = info.num_cores * info.num_subcores
    P_C = 4 if d <= 64 else 2
    CH_E = P_C * K
    EPW = E // NW
    n_chunks = EPW // CH_E
    return NW, P_C, CH_E, EPW, n_chunks


def _graph_feature_sc(x, idxf):
    B, d, N = x.shape
    xt = jnp.transpose(x, (0, 2, 1)).reshape(B * N, d)
    dp = ((d + 127) // 128) * 128
    if dp != d:
        xt = jnp.pad(xt, ((0, 0), (0, dp - d)))
    E = B * N * K
    R = B * N
    NW, P_C, CH_E, EPW, n_chunks = _sc_chunk_meta(R, d, E)
    nb = idxf.reshape(E).reshape(NW, n_chunks, CH_E)
    base_p = (jnp.arange(NW) * EPW)[:, None] + jnp.arange(n_chunks)[None, :] * CH_E
    ctr = jnp.minimum(base_p[:, :, None] // K + jnp.arange(16)[None, None, :], R - 1)
    idx_aug = jnp.concatenate([nb, ctr.astype(jnp.int32)], axis=-1).reshape(-1)
    feat = _sc_feat_fn(R, d, E)(xt, idx_aug)
    return feat.reshape(B, N, K, 2 * d)


def _conv_bn_lrelu(y, W, g, b):
    y = jnp.einsum('...c,oc->...o', y, W)
    axes = tuple(range(y.ndim - 1))
    mu = jnp.mean(y, axis=axes, keepdims=True)
    var = jnp.var(y, axis=axes, keepdims=True)
    y = (y - mu) / jnp.sqrt(var + EPS)
    y = y * g + b
    return jnp.where(y > 0, y, 0.2 * y)


def _graph_feature_flat(x, idxf):
    B, d, N = x.shape
    xt = jnp.transpose(x, (0, 2, 1)).reshape(B * N, d)
    nbrs = xt[idxf]
    center = xt[:, None, :]
    feat = jnp.concatenate([nbrs - center, jnp.broadcast_to(center, nbrs.shape)],
                           axis=-1)
    return feat.reshape(B, N, K, 2 * d)


def _edge_conv(x, W, g, b, use_sc=False):
    idxf = _knn_flat(x)
    if use_sc:
        feat = _graph_feature_sc(x, idxf)
    else:
        feat = _graph_feature_flat(x, idxf)
    y = _conv_bn_lrelu(feat, W, g, b)
    y = jnp.max(y, axis=2)
    return jnp.transpose(y, (0, 2, 1))


def _multi_edge_conv(x, layers):
    idxf = _knn_flat(x)
    feat = _graph_feature_flat(x, idxf)
    for (W, g, b) in layers:
        feat = _conv_bn_lrelu(feat, W, g, b)
    y = jnp.max(feat, axis=2)
    return jnp.transpose(y, (0, 2, 1))


def _encoder_fn(x, p):
    x1 = _multi_edge_conv(x, [(p['enc0_W0'], p['enc0_g0'], p['enc0_b0']),
                              (p['enc0_W1'], p['enc0_g1'], p['enc0_b1'])])
    x2 = _multi_edge_conv(x1, [(p['enc1_W0'], p['enc1_g0'], p['enc1_b0']),
                               (p['enc1_W1'], p['enc1_g1'], p['enc1_b1'])])
    x3 = _edge_conv(x2, p['enc2_W'], p['enc2_g'], p['enc2_b'])
    x4 = _edge_conv(x3, p['enc3_W'], p['enc3_g'], p['enc3_b'])
    return jnp.concatenate([x1, x2, x3, x4], axis=1)


def _tail_fn(x, p):
    y = jnp.einsum('bcn,oc->bon', x, p['tail_W'])
    mu = jnp.mean(y, axis=(0, 2), keepdims=True)
    var = jnp.var(y, axis=(0, 2), keepdims=True)
    y = (y - mu) / jnp.sqrt(var + EPS)
    y = y * p['tail_g'][None, :, None] + p['tail_b'][None, :, None]
    return jnp.where(y > 0, y, 0.2 * y)


def kernel(x, y, params):
    p = params
    x1 = _tail_fn(_encoder_fn(x, p), p)
    x2 = _tail_fn(_encoder_fn(y, p), p)
    z = jnp.concatenate([x1, x2], axis=1)
    z1 = _edge_conv(z, p['dec0_W'], p['dec0_g'], p['dec0_b'])
    z2 = _edge_conv(z1, p['dec1_W'], p['dec1_g'], p['dec1_b'], use_sc=True)
    return jnp.einsum('bcn,oc->bon', z2, p['dec2_W']) + p['dec2_bias'][None, :, None]
